# baseline (speedup 1.0000x reference)
import functools
import math

import jax
import jax.numpy as jnp
from jax import lax
from jax.experimental import pallas as pl
from jax.experimental.pallas import tpu as pltpu
from jax.experimental.pallas import tpu_sc as plsc

VOCAB = 1000000
DIM = 64
BATCH = 4096
SEQ = 200
SCALE = math.sqrt(DIM)

NC = 2
NS = 16
NW = NC * NS
LANES = 16
KSUB = DIM // LANES

R1 = 400
NP1 = VOCAB // R1
ITERS1 = 80

BBLK = 512
NE = BATCH // BBLK
N_UNITS = SEQ * NE
UNITS_PW = N_UNITS // NW
GSUB = 256
NG = BBLK // GSUB


def _iota16():
    return lax.iota(jnp.int32, LANES)


@functools.partial(
    pl.kernel,
    out_type=jax.ShapeDtypeStruct((VOCAB, DIM), jnp.float32),
    mesh=plsc.VectorSubcoreMesh(core_axis_name="c", subcore_axis_name="s"),
    scratch_types=[
        [pltpu.VMEM((DIM, R1), jnp.float32) for _ in range(2)],
        [pltpu.VMEM((R1, DIM), jnp.float32) for _ in range(2)],
        [pltpu.SemaphoreType.DMA for _ in range(2)],
        [pltpu.SemaphoreType.DMA for _ in range(2)],
    ],
    compiler_params=pltpu.CompilerParams(use_tc_tiling_on_sc=False,
                                         needs_layout_passes=False),
)
def _transpose_scale(tt_hbm, out_hbm, ibufs, obufs, isems, osems):
    wid = lax.axis_index("s") * NC + lax.axis_index("c")

    def start_in(c, slot):
        pltpu.async_copy(
            tt_hbm.at[:, pl.ds(c * R1, R1)], ibufs[slot], isems[slot]
        )

    def wait_in(slot):
        pltpu.make_async_copy(
            tt_hbm.at[:, pl.ds(0, R1)], ibufs[slot], isems[slot]
        ).wait()

    def start_out(c, slot):
        pltpu.async_copy(
            obufs[slot], out_hbm.at[pl.ds(c * R1, R1)], osems[slot]
        )

    def wait_out(slot):
        pltpu.make_async_copy(
            obufs[slot], out_hbm.at[pl.ds(0, R1)], osems[slot]
        ).wait()

    iotas = [_iota16() + k * LANES for k in range(KSUB)]

    def transpose_panel(slot):
        ib = ibufs[slot]
        ob = obufs[slot]

        def rows_body(r4, c):
            for rr in range(4):
                r = r4 * 4 + rr
                col = jnp.full((LANES,), 0, jnp.int32) + r
                for k in range(KSUB):
                    v = plsc.load_gather(ib, [iotas[k], col])
                    ob[r, pl.ds(k * LANES, LANES)] = v * SCALE
            return c

        lax.fori_loop(0, R1 // 4, rows_body, 0)

    @pl.when(wid < NP1)
    def _():
        start_in(wid, 0)

    def outer(k2, carry):
        for p in range(2):
            k = k2 * 2 + p
            c = k * NW + wid
            cn = c + NW
            slot = p
            nslot = 1 - p

            @pl.when(cn < NP1)
            def _():
                start_in(cn, nslot)

            @pl.when(c < NP1)
            def _():
                wait_in(slot)

                @pl.when(k >= 2)
                def _():
                    wait_out(slot)

                transpose_panel(slot)
                start_out(c, slot)

        return carry

    lax.fori_loop(0, ITERS1 // 2, outer, 0)

    for j in range(ITERS1 - 4, ITERS1):
        c = j * NW + wid

        @pl.when(jnp.logical_and(c < NP1, c + 2 * NW >= NP1))
        def _(j=j):
            wait_out(j % 2)


@functools.partial(
    pl.kernel,
    out_type=jax.ShapeDtypeStruct((SEQ, DIM, BATCH), jnp.float32),
    mesh=plsc.VectorSubcoreMesh(core_axis_name="c", subcore_axis_name="s"),
    scratch_types=[
        [pltpu.VMEM((BBLK,), jnp.int32) for _ in range(2)],
        [pltpu.VMEM((GSUB, DIM), jnp.float32) for _ in range(2)],
        [pltpu.VMEM((DIM, BBLK), jnp.float32) for _ in range(2)],
        [pltpu.SemaphoreType.DMA for _ in range(2)],
        [pltpu.SemaphoreType.DMA for _ in range(2)],
        [pltpu.SemaphoreType.DMA for _ in range(2)],
    ],
    compiler_params=pltpu.CompilerParams(use_tc_tiling_on_sc=False,
                                         needs_layout_passes=False),
)
def _gather_t(xt_hbm, table_hbm, out_hbm, idxbufs, abufs, bbufs,
              isems, gsems, osems):
    wid = lax.axis_index("s") * NC + lax.axis_index("c")

    def unit_su(u):
        return u // NE, u % NE

    def start_idx(u, slot):
        s, e = unit_su(u)
        pltpu.async_copy(
            xt_hbm.at[s, pl.ds(e * BBLK, BBLK)], idxbufs[slot], isems[slot]
        )

    def wait_idx(slot):
        pltpu.make_async_copy(
            xt_hbm.at[0, pl.ds(0, BBLK)], idxbufs[slot], isems[slot]
        ).wait()

    def start_gather(islot, sub, aslot):
        pltpu.async_copy(
            table_hbm.at[idxbufs[islot].at[pl.ds(sub * GSUB, GSUB)]],
            abufs[aslot],
            gsems[aslot],
        )

    def wait_gather(islot, aslot):
        pltpu.make_async_copy(
            table_hbm.at[idxbufs[islot].at[pl.ds(0, GSUB)]],
            abufs[aslot],
            gsems[aslot],
        ).wait()

    def start_out(u, bslot):
        s, e = unit_su(u)
        pltpu.async_copy(
            bbufs[bslot], out_hbm.at[s, :, pl.ds(e * BBLK, BBLK)], osems[bslot]
        )

    def wait_out(bslot):
        pltpu.make_async_copy(
            bbufs[bslot], out_hbm.at[0, :, pl.ds(0, BBLK)], osems[bslot]
        ).wait()

    iotas = [_iota16() + k * LANES for k in range(KSUB)]

    def transpose_block(aslot, bslot, sub):
        ab = abufs[aslot]
        bb = bbufs[bslot]
        base = sub * GSUB

        def rows_body(r4, c):
            for rr in range(4):
                r = r4 * 4 + rr
                col = jnp.full((LANES,), base, jnp.int32) + r
                for k in range(KSUB):
                    v = ab[r, pl.ds(k * LANES, LANES)]
                    plsc.store_scatter(bb, [iotas[k], col], v)
            return c

        lax.fori_loop(0, GSUB // 4, rows_body, 0)

    u0 = wid * UNITS_PW
    start_idx(u0, 0)
    wait_idx(0)
    start_gather(0, 0, 0)

    def outer(k2, carry):
        for p in range(2):
            k = k2 * 2 + p
            u = u0 + k
            islot = p
            bslot = p

            @pl.when(k + 1 < UNITS_PW)
            def _():
                start_idx(u + 1, 1 - p)

            @pl.when(k >= 2)
            def _():
                wait_out(bslot)

            for sub in range(NG):
                aslot = sub % 2
                if sub + 1 < NG:
                    start_gather(islot, sub + 1, (sub + 1) % 2)
                else:
                    @pl.when(k + 1 < UNITS_PW)
                    def _():
                        wait_idx(1 - p)
                        start_gather(1 - p, 0, (sub + 1) % 2)

                wait_gather(islot, aslot)
                transpose_block(aslot, bslot, sub)

            start_out(u, bslot)
        return carry

    lax.fori_loop(0, UNITS_PW // 2, outer, 0)

    for j in (UNITS_PW - 2, UNITS_PW - 1):
        wait_out(j % 2)


def kernel(x, table):
    xt = x.T.astype(jnp.int32)
    tt = table.T
    t_rm = _transpose_scale(tt)
    out_t = _gather_t(xt, t_rm)
    return out_t.transpose(2, 0, 1)


# speedup vs baseline: 3.7764x; 3.7764x over previous
import functools
import math

import jax
import jax.numpy as jnp
from jax import lax
from jax.experimental import pallas as pl
from jax.experimental.pallas import tpu as pltpu
from jax.experimental.pallas import tpu_sc as plsc

VOCAB = 1000000
DIM = 64
BATCH = 4096
SEQ = 200
SCALE = math.sqrt(DIM)

NC = 2
NS = 16
NW = NC * NS
LANES = 16
KSUB = DIM // LANES

R1 = 400
NP1 = VOCAB // R1
ITERS1 = 80

BBLK = 512
NE = BATCH // BBLK
N_UNITS = SEQ * NE
UNITS_PW = N_UNITS // NW
GSUB = 256
NG = BBLK // GSUB


def _iota16():
    return lax.iota(jnp.int32, LANES)


@functools.partial(
    pl.kernel,
    out_type=jax.ShapeDtypeStruct((VOCAB, DIM), jnp.float32),
    mesh=plsc.VectorSubcoreMesh(core_axis_name="c", subcore_axis_name="s"),
    scratch_types=[
        [pltpu.VMEM((DIM, R1), jnp.float32) for _ in range(2)],
        [pltpu.VMEM((R1, DIM), jnp.float32) for _ in range(2)],
        [pltpu.SemaphoreType.DMA for _ in range(2)],
        [pltpu.SemaphoreType.DMA for _ in range(2)],
    ],
    compiler_params=pltpu.CompilerParams(use_tc_tiling_on_sc=False,
                                         needs_layout_passes=False),
)
def _transpose_scale(tt_hbm, out_hbm, ibufs, obufs, isems, osems):
    wid = lax.axis_index("s") * NC + lax.axis_index("c")

    def start_in(c, slot):
        pltpu.async_copy(
            tt_hbm.at[:, pl.ds(c * R1, R1)], ibufs[slot], isems[slot]
        )

    def wait_in(slot):
        pltpu.make_async_copy(
            tt_hbm.at[:, pl.ds(0, R1)], ibufs[slot], isems[slot]
        ).wait()

    def start_out(c, slot):
        pltpu.async_copy(
            obufs[slot], out_hbm.at[pl.ds(c * R1, R1)], osems[slot]
        )

    def wait_out(slot):
        pltpu.make_async_copy(
            obufs[slot], out_hbm.at[pl.ds(0, R1)], osems[slot]
        ).wait()

    iotas = [_iota16() + k * LANES for k in range(KSUB)]

    def transpose_panel(slot):
        ib = ibufs[slot]
        ob = obufs[slot]

        def rows_body(r4, c):
            for rr in range(4):
                r = r4 * 4 + rr
                col = jnp.full((LANES,), 0, jnp.int32) + r
                for k in range(KSUB):
                    v = plsc.load_gather(ib, [iotas[k], col])
                    ob[r, pl.ds(k * LANES, LANES)] = v * SCALE
            return c

        lax.fori_loop(0, R1 // 4, rows_body, 0)

    @pl.when(wid < NP1)
    def _():
        start_in(wid, 0)

    def outer(k2, carry):
        for p in range(2):
            k = k2 * 2 + p
            c = k * NW + wid
            cn = c + NW
            slot = p
            nslot = 1 - p

            @pl.when(cn < NP1)
            def _():
                start_in(cn, nslot)

            @pl.when(c < NP1)
            def _():
                wait_in(slot)

                @pl.when(k >= 2)
                def _():
                    wait_out(slot)

                transpose_panel(slot)
                start_out(c, slot)

        return carry

    lax.fori_loop(0, ITERS1 // 2, outer, 0)

    for j in range(ITERS1 - 4, ITERS1):
        c = j * NW + wid

        @pl.when(jnp.logical_and(c < NP1, c + 2 * NW >= NP1))
        def _(j=j):
            wait_out(j % 2)


@functools.partial(
    pl.kernel,
    out_type=jax.ShapeDtypeStruct((SEQ, DIM, BATCH), jnp.float32),
    mesh=plsc.VectorSubcoreMesh(core_axis_name="c", subcore_axis_name="s"),
    scratch_types=[
        [pltpu.VMEM((BBLK,), jnp.int32) for _ in range(2)],
        [pltpu.VMEM((GSUB, DIM), jnp.float32) for _ in range(2)],
        [pltpu.VMEM((DIM, BBLK), jnp.float32) for _ in range(2)],
        [pltpu.SemaphoreType.DMA for _ in range(2)],
        [pltpu.SemaphoreType.DMA for _ in range(2)],
        [pltpu.SemaphoreType.DMA for _ in range(2)],
    ],
    compiler_params=pltpu.CompilerParams(use_tc_tiling_on_sc=False,
                                         needs_layout_passes=False),
)
def _gather_t(xt_hbm, table_hbm, out_hbm, idxbufs, abufs, bbufs,
              isems, gsems, osems):
    wid = lax.axis_index("s") * NC + lax.axis_index("c")

    def unit_su(u):
        return u // NE, u % NE

    def start_idx(u, slot):
        s, e = unit_su(u)
        pltpu.async_copy(
            xt_hbm.at[s, pl.ds(e * BBLK, BBLK)], idxbufs[slot], isems[slot]
        )

    def wait_idx(slot):
        pltpu.make_async_copy(
            xt_hbm.at[0, pl.ds(0, BBLK)], idxbufs[slot], isems[slot]
        ).wait()

    def start_gather(islot, sub, aslot):
        pltpu.async_copy(
            table_hbm.at[idxbufs[islot].at[pl.ds(sub * GSUB, GSUB)]],
            abufs[aslot],
            gsems[aslot],
        )

    def wait_gather(islot, aslot):
        pltpu.make_async_copy(
            table_hbm.at[idxbufs[islot].at[pl.ds(0, GSUB)]],
            abufs[aslot],
            gsems[aslot],
        ).wait()

    def start_out(u, bslot):
        s, e = unit_su(u)
        pltpu.async_copy(
            bbufs[bslot], out_hbm.at[s, :, pl.ds(e * BBLK, BBLK)], osems[bslot]
        )

    def wait_out(bslot):
        pltpu.make_async_copy(
            bbufs[bslot], out_hbm.at[0, :, pl.ds(0, BBLK)], osems[bslot]
        ).wait()

    iotas = [_iota16() + k * LANES for k in range(KSUB)]

    def transpose_block(aslot, bslot, sub):
        ab = abufs[aslot]
        bb = bbufs[bslot]
        base = sub * GSUB

        def rows_body(r4, c):
            for rr in range(4):
                r = r4 * 4 + rr
                col = jnp.full((LANES,), base, jnp.int32) + r
                for k in range(KSUB):
                    v = ab[r, pl.ds(k * LANES, LANES)] * SCALE
                    plsc.store_scatter(bb, [iotas[k], col], v)
            return c

        lax.fori_loop(0, GSUB // 4, rows_body, 0)

    u0 = wid * UNITS_PW
    start_idx(u0, 0)
    wait_idx(0)
    start_gather(0, 0, 0)

    def outer(k2, carry):
        for p in range(2):
            k = k2 * 2 + p
            u = u0 + k
            islot = p
            bslot = p

            @pl.when(k + 1 < UNITS_PW)
            def _():
                start_idx(u + 1, 1 - p)

            @pl.when(k >= 2)
            def _():
                wait_out(bslot)

            for sub in range(NG):
                aslot = sub % 2
                if sub + 1 < NG:
                    start_gather(islot, sub + 1, (sub + 1) % 2)
                else:
                    @pl.when(k + 1 < UNITS_PW)
                    def _():
                        wait_idx(1 - p)
                        start_gather(1 - p, 0, (sub + 1) % 2)

                wait_gather(islot, aslot)
                transpose_block(aslot, bslot, sub)

            start_out(u, bslot)
        return carry

    lax.fori_loop(0, UNITS_PW // 2, outer, 0)

    for j in (UNITS_PW - 2, UNITS_PW - 1):
        wait_out(j % 2)


def kernel(x, table):
    xt = x.T.astype(jnp.int32)
    out_t = _gather_t(xt, table)
    return out_t.transpose(2, 0, 1)


# speedup vs baseline: 5.5391x; 1.4668x over previous
import functools
import math

import jax
import jax.numpy as jnp
from jax import lax
from jax.experimental import pallas as pl
from jax.experimental.pallas import tpu as pltpu
from jax.experimental.pallas import tpu_sc as plsc

VOCAB = 1000000
DIM = 64
BATCH = 4096
SEQ = 200
SCALE = math.sqrt(DIM)

NC = 2
NS = 16
NW = NC * NS
LANES = 16
KSUB = DIM // LANES

R1 = 400
NP1 = VOCAB // R1
ITERS1 = 80

BBLK = 512
NE = BATCH // BBLK
N_UNITS = SEQ * NE
UNITS_PW = N_UNITS // NW
GSUB = 256
NG = BBLK // GSUB


def _iota16():
    return lax.iota(jnp.int32, LANES)


@functools.partial(
    pl.kernel,
    out_type=jax.ShapeDtypeStruct((VOCAB, DIM), jnp.float32),
    mesh=plsc.VectorSubcoreMesh(core_axis_name="c", subcore_axis_name="s"),
    scratch_types=[
        [pltpu.VMEM((DIM, R1), jnp.float32) for _ in range(2)],
        [pltpu.VMEM((R1, DIM), jnp.float32) for _ in range(2)],
        [pltpu.SemaphoreType.DMA for _ in range(2)],
        [pltpu.SemaphoreType.DMA for _ in range(2)],
    ],
    compiler_params=pltpu.CompilerParams(use_tc_tiling_on_sc=False,
                                         needs_layout_passes=False),
)
def _transpose_scale(tt_hbm, out_hbm, ibufs, obufs, isems, osems):
    wid = lax.axis_index("s") * NC + lax.axis_index("c")

    def start_in(c, slot):
        pltpu.async_copy(
            tt_hbm.at[:, pl.ds(c * R1, R1)], ibufs[slot], isems[slot]
        )

    def wait_in(slot):
        pltpu.make_async_copy(
            tt_hbm.at[:, pl.ds(0, R1)], ibufs[slot], isems[slot]
        ).wait()

    def start_out(c, slot):
        pltpu.async_copy(
            obufs[slot], out_hbm.at[pl.ds(c * R1, R1)], osems[slot]
        )

    def wait_out(slot):
        pltpu.make_async_copy(
            obufs[slot], out_hbm.at[pl.ds(0, R1)], osems[slot]
        ).wait()

    iotas = [_iota16() + k * LANES for k in range(KSUB)]

    def transpose_panel(slot):
        ib = ibufs[slot]
        ob = obufs[slot]

        def rows_body(r4, c):
            for rr in range(4):
                r = r4 * 4 + rr
                col = jnp.full((LANES,), 0, jnp.int32) + r
                for k in range(KSUB):
                    v = plsc.load_gather(ib, [iotas[k], col])
                    ob[r, pl.ds(k * LANES, LANES)] = v * SCALE
            return c

        lax.fori_loop(0, R1 // 4, rows_body, 0)

    @pl.when(wid < NP1)
    def _():
        start_in(wid, 0)

    def outer(k2, carry):
        for p in range(2):
            k = k2 * 2 + p
            c = k * NW + wid
            cn = c + NW
            slot = p
            nslot = 1 - p

            @pl.when(cn < NP1)
            def _():
                start_in(cn, nslot)

            @pl.when(c < NP1)
            def _():
                wait_in(slot)

                @pl.when(k >= 2)
                def _():
                    wait_out(slot)

                transpose_panel(slot)
                start_out(c, slot)

        return carry

    lax.fori_loop(0, ITERS1 // 2, outer, 0)

    for j in range(ITERS1 - 4, ITERS1):
        c = j * NW + wid

        @pl.when(jnp.logical_and(c < NP1, c + 2 * NW >= NP1))
        def _(j=j):
            wait_out(j % 2)


@functools.partial(
    pl.kernel,
    out_type=jax.ShapeDtypeStruct((SEQ, DIM, BATCH), jnp.float32),
    mesh=plsc.VectorSubcoreMesh(core_axis_name="c", subcore_axis_name="s"),
    scratch_types=[
        [pltpu.VMEM((BBLK,), jnp.int32) for _ in range(2)],
        [pltpu.VMEM((GSUB, DIM), jnp.float32) for _ in range(2)],
        [pltpu.VMEM((DIM, BBLK + 1), jnp.float32) for _ in range(2)],
        [pltpu.SemaphoreType.DMA for _ in range(2)],
        [pltpu.SemaphoreType.DMA for _ in range(2)],
        [pltpu.SemaphoreType.DMA for _ in range(2)],
    ],
    compiler_params=pltpu.CompilerParams(use_tc_tiling_on_sc=False,
                                         needs_layout_passes=False),
)
def _gather_t(xt_hbm, table_hbm, out_hbm, idxbufs, abufs, bbufs,
              isems, gsems, osems):
    wid = lax.axis_index("s") * NC + lax.axis_index("c")

    def unit_su(u):
        return u // NE, u % NE

    def start_idx(u, slot):
        s, e = unit_su(u)
        pltpu.async_copy(
            xt_hbm.at[s, pl.ds(e * BBLK, BBLK)], idxbufs[slot], isems[slot]
        )

    def wait_idx(slot):
        pltpu.make_async_copy(
            xt_hbm.at[0, pl.ds(0, BBLK)], idxbufs[slot], isems[slot]
        ).wait()

    def start_gather(islot, sub, aslot):
        pltpu.async_copy(
            table_hbm.at[idxbufs[islot].at[pl.ds(sub * GSUB, GSUB)]],
            abufs[aslot],
            gsems[aslot],
        )

    def wait_gather(islot, aslot):
        pltpu.make_async_copy(
            table_hbm.at[idxbufs[islot].at[pl.ds(0, GSUB)]],
            abufs[aslot],
            gsems[aslot],
        ).wait()

    def start_out(u, bslot):
        s, e = unit_su(u)
        pltpu.async_copy(
            bbufs[bslot].at[:, pl.ds(0, BBLK)],
            out_hbm.at[s, :, pl.ds(e * BBLK, BBLK)],
            osems[bslot],
        )

    def wait_out(bslot):
        pltpu.make_async_copy(
            bbufs[bslot].at[:, pl.ds(0, BBLK)],
            out_hbm.at[0, :, pl.ds(0, BBLK)],
            osems[bslot],
        ).wait()

    iotas = [_iota16() + k * LANES for k in range(KSUB)]

    def transpose_block(aslot, bslot, sub):
        ab = abufs[aslot]
        bb = bbufs[bslot]
        base = sub * GSUB

        def rows_body(r4, c):
            for rr in range(4):
                r = r4 * 4 + rr
                col = jnp.full((LANES,), base, jnp.int32) + r
                for k in range(KSUB):
                    v = ab[r, pl.ds(k * LANES, LANES)] * SCALE
                    plsc.store_scatter(bb, [iotas[k], col], v)
            return c

        lax.fori_loop(0, GSUB // 4, rows_body, 0)

    u0 = wid * UNITS_PW
    start_idx(u0, 0)
    wait_idx(0)
    start_gather(0, 0, 0)

    def outer(k2, carry):
        for p in range(2):
            k = k2 * 2 + p
            u = u0 + k
            islot = p
            bslot = p

            @pl.when(k + 1 < UNITS_PW)
            def _():
                start_idx(u + 1, 1 - p)

            @pl.when(k >= 2)
            def _():
                wait_out(bslot)

            for sub in range(NG):
                aslot = sub % 2
                if sub + 1 < NG:
                    start_gather(islot, sub + 1, (sub + 1) % 2)
                else:
                    @pl.when(k + 1 < UNITS_PW)
                    def _():
                        wait_idx(1 - p)
                        start_gather(1 - p, 0, (sub + 1) % 2)

                wait_gather(islot, aslot)
                transpose_block(aslot, bslot, sub)

            start_out(u, bslot)
        return carry

    lax.fori_loop(0, UNITS_PW // 2, outer, 0)

    for j in (UNITS_PW - 2, UNITS_PW - 1):
        wait_out(j % 2)


def kernel(x, table):
    xt = x.T.astype(jnp.int32)
    out_t = _gather_t(xt, table)
    return out_t.transpose(2, 0, 1)
